# Initial kernel scaffold; baseline (speedup 1.0000x reference)
import jax
import jax.numpy as jnp
from jax.experimental import pallas as pl


def kernel(x, edge_index, edge_weight, h, Wxz, bxz, Whz, bhz, Wxr, bxr, Whr, bhr, Wxh, bxh, Whh, bhh, W1, b1, W2, b2):
    raise NotImplementedError("write your pallas kernel here")


# speedup vs baseline: 3.2750x; 3.2750x over previous
import jax
import jax.numpy as jnp
from jax.experimental import pallas as pl


def _gru_body(x_ref, h0_ref, wx_ref, bx_ref, whzr_ref, bhzr_ref, whh_ref,
              bhh_ref, w1_ref, b1_ref, w2_ref, b2_ref, out_ref, hT_ref):
    i = pl.program_id(0)
    T = x_ref.shape[0]
    H = h0_ref.shape[1]

    @pl.when(i == 0)
    def _init():
        out_ref[...] = jnp.full(out_ref.shape, b2_ref[0, 0], dtype=jnp.float32)

    h = h0_ref[...]
    wx = wx_ref[...]
    whzr = whzr_ref[...]
    whh = whh_ref[...]
    bx = bx_ref[...]
    bhzr = bhzr_ref[...]
    bhh = bhh_ref[...]
    w1 = w1_ref[...]
    b1 = b1_ref[0, 0]
    w2 = w2_ref[...]

    for t in range(T):
        xt = x_ref[t]
        xp = jnp.dot(xt, wx, preferred_element_type=jnp.float32) + bx
        hp = jnp.dot(h, whzr, preferred_element_type=jnp.float32) + bhzr
        z = jax.nn.sigmoid(xp[:, :H] + hp[:, :H])
        r = jax.nn.sigmoid(xp[:, H:2 * H] + hp[:, H:2 * H])
        ht = jnp.tanh(xp[:, 2 * H:] +
                      jnp.dot(h * r, whh, preferred_element_type=jnp.float32) +
                      bhh)
        h = z * h + (1.0 - z) * ht
        hh1 = jnp.where(h >= 0, h, 0.01 * h)
        v = jnp.dot(hh1, w1, preferred_element_type=jnp.float32) + b1
        hh2 = jnp.where(v >= 0, v, 0.01 * v)
        s = jnp.sum(hh2 * w2)
        out_ref[t, :] = out_ref[t, :] + s

    hT_ref[...] = h


def kernel(x, edge_index, edge_weight, h, Wxz, bxz, Whz, bhz, Wxr, bxr, Whr,
           bhr, Wxh, bxh, Whh, bhh, W1, b1, W2, b2):
    T, N, F = x.shape
    H = h.shape[1]

    Wx = jnp.concatenate([Wxz, Wxr, Wxh], axis=1)
    bx = jnp.concatenate([bxz, bxr, bxh]).reshape(1, 3 * H)
    Whzr = jnp.concatenate([Whz, Whr], axis=1)
    bhzr = jnp.concatenate([bhz, bhr]).reshape(1, 2 * H)
    bhh2 = bhh.reshape(1, H)
    b1r = b1.reshape(1, 1)
    b2r = b2.reshape(1, 1)

    B = next(b for b in (2000, 1000, 500, 200, 100, 40, 8, 1) if N % b == 0)
    grid = (N // B,)

    full = lambda shape: pl.BlockSpec(shape, lambda i: (0,) * len(shape))

    out_acc, hT = pl.pallas_call(
        _gru_body,
        grid=grid,
        in_specs=[
            pl.BlockSpec((T, B, F), lambda i: (0, i, 0)),
            pl.BlockSpec((B, H), lambda i: (i, 0)),
            full((F, 3 * H)),
            full((1, 3 * H)),
            full((H, 2 * H)),
            full((1, 2 * H)),
            full((H, H)),
            full((1, H)),
            full((H, 1)),
            full((1, 1)),
            pl.BlockSpec((B, 1), lambda i: (i, 0)),
            full((1, 1)),
        ],
        out_specs=[
            pl.BlockSpec((T, 128), lambda i: (0, 0)),
            pl.BlockSpec((B, H), lambda i: (i, 0)),
        ],
        out_shape=[
            jax.ShapeDtypeStruct((T, 128), jnp.float32),
            jax.ShapeDtypeStruct((N, H), jnp.float32),
        ],
    )(x, h, Wx, bx, Whzr, bhzr, Whh, bhh2, W1, b1r, W2, b2r)

    return out_acc[:, 0], hT
